# Initial kernel scaffold; baseline (speedup 1.0000x reference)
import jax
import jax.numpy as jnp
from jax.experimental import pallas as pl


def kernel(x, edge_index, pos_edge_index, neg_edge_index, W_self_0, W_neigh_0, b_0, W_self_1, W_neigh_1, b_1, W_self_2, W_neigh_2, b_2):
    raise NotImplementedError("write your pallas kernel here")


# speedup vs baseline: 2.7097x; 2.7097x over previous
import functools

import jax
import jax.numpy as jnp
from jax import lax
from jax.experimental import pallas as pl
from jax.experimental.pallas import tpu as pltpu
from jax.experimental.pallas import tpu_sc as plsc

NC = 2
NS = 16
NW = NC * NS
LANES = 16


EC = 128


@functools.partial(jax.jit, static_argnames=("n", "d", "nch"))
def _agg_call(h, src3d, dst3d, *, n, d, nch):
    nfull = n // EC
    ntail = n - nfull * EC
    kmax = (nfull + NS - 1) // NS

    mesh = plsc.VectorSubcoreMesh(core_axis_name="c", subcore_axis_name="s")

    scratch = [
        pltpu.VMEM((nch, EC), jnp.int32),
        pltpu.VMEM((nch, EC), jnp.int32),
        pltpu.VMEM((EC, d), jnp.float32),
        pltpu.VMEM_SHARED((n + LANES, d), jnp.float32),
        pltpu.SemaphoreType.DMA,
    ]

    def body(h_hbm, src_hbm, dst_hbm, agg_out, src_v, dst_v, rows_v, agg_sh, sem):
        cid = lax.axis_index("c")
        sid = lax.axis_index("s")
        wid = sid * NC + cid

        zeros = jnp.zeros((LANES,), jnp.float32)

        def zfill(i, _):
            for s in range(d // LANES):
                rows_v[i, pl.ds(s * LANES, LANES)] = zeros
            return 0
        lax.fori_loop(0, EC, zfill, 0)

        for k in range(kmax):
            idx = k * NS + sid
            @pl.when(idx < nfull)
            def _():
                pltpu.sync_copy(rows_v, agg_sh.at[pl.ds(idx * EC, EC)])
        if ntail:
            @pl.when(sid == NS - 1)
            def _():
                pltpu.sync_copy(rows_v.at[pl.ds(0, ntail)],
                                agg_sh.at[pl.ds(nfull * EC, ntail)])
        plsc.subcore_barrier()

        pltpu.sync_copy(src_hbm.at[wid], src_v)
        pltpu.sync_copy(dst_hbm.at[wid], dst_v)

        def step(j, _):
            pltpu.async_copy(h_hbm.at[src_v.at[j]], rows_v, sem).wait()
            pltpu.sync_copy(rows_v, agg_sh.at[dst_v.at[j]], add=True)
            return 0
        lax.fori_loop(0, nch, step, 0)

        plsc.subcore_barrier()

        for k in range(kmax):
            idx = k * NS + sid
            @pl.when(idx < nfull)
            def _():
                r0 = idx * EC
                pltpu.sync_copy(agg_sh.at[pl.ds(r0, EC)], rows_v)
                pltpu.sync_copy(rows_v, agg_out.at[cid, pl.ds(r0, EC)])
        if ntail:
            @pl.when(sid == NS - 1)
            def _():
                r0 = nfull * EC
                pltpu.sync_copy(agg_sh.at[pl.ds(r0, ntail)], rows_v.at[pl.ds(0, ntail)])
                pltpu.sync_copy(rows_v.at[pl.ds(0, ntail)], agg_out.at[cid, pl.ds(r0, ntail)])

    fn = pl.kernel(body,
                   out_type=jax.ShapeDtypeStruct((NC, n, d), jnp.float32),
                   mesh=mesh, scratch_types=scratch)
    return fn(h, src3d, dst3d)


def _layer_body(h_ref, aggp_ref, degp_ref, ws_ref, wn_ref, b_ref, out_ref):
    agg = aggp_ref[0] + aggp_ref[1]
    deg = degp_ref[0, :, 0] + degp_ref[1, :, 0]
    invd = 1.0 / jnp.maximum(deg, 1.0)
    m = agg * invd[:, None]
    z = (jnp.dot(h_ref[...], ws_ref[...], preferred_element_type=jnp.float32)
         + jnp.dot(m, wn_ref[...], preferred_element_type=jnp.float32)
         + b_ref[...])
    z = jnp.where(z >= 0, z, 0.2 * z)
    nrm = jnp.sqrt(jnp.sum(z * z, axis=-1, keepdims=True)) + 1e-12
    out_ref[...] = z / nrm


@functools.partial(jax.jit, static_argnames=("n", "d", "bn"))
def _layer_call(h, aggp, degp, ws, wn, b, *, n, d, bn):
    grid = (n // bn,)
    return pl.pallas_call(
        _layer_body,
        grid=grid,
        in_specs=[
            pl.BlockSpec((bn, d), lambda i: (i, 0)),
            pl.BlockSpec((NC, bn, d), lambda i: (0, i, 0)),
            pl.BlockSpec((NC, bn, d), lambda i: (0, i, 0)),
            pl.BlockSpec((d, d), lambda i: (0, 0)),
            pl.BlockSpec((d, d), lambda i: (0, 0)),
            pl.BlockSpec((1, d), lambda i: (0, 0)),
        ],
        out_specs=pl.BlockSpec((bn, d), lambda i: (i, 0)),
        out_shape=jax.ShapeDtypeStruct((n, d), jnp.float32),
    )(h, aggp, degp, ws, wn, b.reshape(1, d))


@functools.partial(jax.jit, static_argnames=("n", "d", "ptot"))
def _scores_call(h, a2d, b2d, *, n, d, ptot):
    CS = 128
    ppw = ptot // NW
    nch = ppw // CS
    assert ppw * NW == ptot and nch * CS == ppw

    mesh = plsc.VectorSubcoreMesh(core_axis_name="c", subcore_axis_name="s")

    def body(h_hbm, a_hbm, b_hbm, out_hbm, a_v, b_v, ra_v, rb_v, out_v, sa, sb):
        cid = lax.axis_index("c")
        sid = lax.axis_index("s")
        wid = sid * NC + cid

        pltpu.sync_copy(a_hbm.at[wid], a_v)
        pltpu.sync_copy(b_hbm.at[wid], b_v)

        def chunk(j, _):
            cpa = pltpu.async_copy(h_hbm.at[a_v.at[j]], ra_v, sa)
            cpb = pltpu.async_copy(h_hbm.at[b_v.at[j]], rb_v, sb)
            cpa.wait()
            cpb.wait()

            def pair(p, _):
                acc = ra_v[p, pl.ds(0, LANES)] * rb_v[p, pl.ds(0, LANES)]
                for s in range(1, d // LANES):
                    acc = acc + (ra_v[p, pl.ds(s * LANES, LANES)]
                                 * rb_v[p, pl.ds(s * LANES, LANES)])
                out_v[p, pl.ds(0, LANES)] = acc
                return 0
            lax.fori_loop(0, CS, pair, 0)
            pltpu.sync_copy(out_v, out_hbm.at[pl.ds((wid * nch + j) * CS, CS)])
            return 0
        lax.fori_loop(0, nch, chunk, 0)

    fn = pl.kernel(
        body,
        out_type=jax.ShapeDtypeStruct((ptot, LANES), jnp.float32),
        mesh=mesh,
        scratch_types=[
            pltpu.VMEM((nch, CS), jnp.int32),
            pltpu.VMEM((nch, CS), jnp.int32),
            pltpu.VMEM((CS, d), jnp.float32),
            pltpu.VMEM((CS, d), jnp.float32),
            pltpu.VMEM((CS, LANES), jnp.float32),
            pltpu.SemaphoreType.DMA,
            pltpu.SemaphoreType.DMA,
        ],
    )
    return fn(h, a2d, b2d)


def _score_reduce_body(part_ref, out_ref):
    out_ref[...] = jnp.sum(part_ref[...], axis=-1)


@functools.partial(jax.jit, static_argnames=("ptot", "bp"))
def _score_reduce(part, *, ptot, bp):
    return pl.pallas_call(
        _score_reduce_body,
        grid=(ptot // bp,),
        in_specs=[pl.BlockSpec((bp, LANES), lambda i: (i, 0))],
        out_specs=pl.BlockSpec((bp,), lambda i: (i,)),
        out_shape=jax.ShapeDtypeStruct((ptot,), jnp.float32),
    )(part)


def kernel(x, edge_index, pos_edge_index, neg_edge_index,
           W_self_0, W_neigh_0, b_0,
           W_self_1, W_neigh_1, b_1,
           W_self_2, W_neigh_2, b_2):
    n, d = x.shape
    e = edge_index.shape[1]
    npair = pos_edge_index.shape[1]

    equant = NW * EC
    epad = ((e + equant - 1) // equant) * equant
    nch = epad // equant
    src_p = jnp.concatenate([edge_index[0], jnp.zeros((epad - e,), jnp.int32)])
    dst_p = jnp.concatenate([edge_index[1], jnp.full((epad - e,), n, jnp.int32)])
    src3d = src_p.reshape(NW, nch, EC)
    dst3d = dst_p.reshape(NW, nch, EC)

    ws = [(W_self_0, W_neigh_0, b_0), (W_self_1, W_neigh_1, b_1), (W_self_2, W_neigh_2, b_2)]
    degp = _agg_call(jnp.ones((n, d), jnp.float32), src3d, dst3d, n=n, d=d, nch=nch)
    h = x
    for (wself, wneigh, bb) in ws:
        aggp = _agg_call(h, src3d, dst3d, n=n, d=d, nch=nch)
        h = _layer_call(h, aggp, degp, wself, wneigh, bb, n=n, d=d, bn=1000)
    h3 = h

    CS = 128
    quant = NW * CS
    npad = ((npair + quant - 1) // quant) * quant
    pz = npad - npair

    def padcat(row):
        return jnp.concatenate([row, jnp.zeros((pz,), jnp.int32)])

    a_all = jnp.concatenate([padcat(pos_edge_index[0]), padcat(neg_edge_index[0])])
    b_all = jnp.concatenate([padcat(pos_edge_index[1]), padcat(neg_edge_index[1])])
    nchs = 2 * npad // (NW * CS)
    part = _scores_call(h3, a_all.reshape(NW, nchs, CS), b_all.reshape(NW, nchs, CS),
                        n=n, d=d, ptot=2 * npad)
    scores = _score_reduce(part, ptot=2 * npad, bp=2048)
    pos_score = scores[:npair]
    neg_score = scores[npad:npad + npair]
    return (h3, pos_score, neg_score)
